# baseline (speedup 1.0000x reference)
import jax
import jax.numpy as jnp
from jax.experimental import pallas as pl
from jax.experimental.pallas import tpu as pltpu


def _moe_block_kernel(x_ref, noise_ref, gate_ref, w_ref, b_ref, out_ref):
    xt = x_ref[...]
    logits = jnp.dot(xt, gate_ref[...], preferred_element_type=jnp.float32)
    g = jax.nn.softmax(logits, axis=-1)
    xp = (xt + noise_ref[...]).astype(jnp.bfloat16)
    e_total = b_ref.shape[0]
    d = xp.shape[1]
    acc = jnp.zeros(out_ref.shape, jnp.float32)
    for e in range(e_total):
        ye = jnp.dot(xp, w_ref[e * d:(e + 1) * d, :],
                     preferred_element_type=jnp.float32)
        acc = acc + g[:, e:e + 1] * (ye + b_ref[e:e + 1, :])
    out_ref[...] = acc


def kernel(x, gate, W, b, noise):
    B, N, D = x.shape
    E = gate.shape[1]
    O = W.shape[2]
    T = B * N
    TT = 1024
    xf = x.reshape(T, D)
    Wb = W.astype(jnp.bfloat16).reshape(E * D, O)
    bb = b.astype(jnp.bfloat16)
    nb = N // TT
    out = pl.pallas_call(
        _moe_block_kernel,
        grid=(T // TT,),
        in_specs=[
            pl.BlockSpec((TT, D), lambda i: (i, 0)),
            pl.BlockSpec((TT, D), lambda i: (i % nb, 0)),
            pl.BlockSpec((D, E), lambda i: (0, 0)),
            pl.BlockSpec((E * D, O), lambda i: (0, 0)),
            pl.BlockSpec((E, O), lambda i: (0, 0)),
        ],
        out_specs=pl.BlockSpec((TT, O), lambda i: (i, 0)),
        out_shape=jax.ShapeDtypeStruct((T, O), jnp.float32),
        compiler_params=pltpu.CompilerParams(
            dimension_semantics=("parallel",),
            allow_input_fusion=[False, False, False, True, True],
        ),
    )(xf, noise, gate, Wb, bb)
    return out.reshape(B, N, O)


# speedup vs baseline: 1.0712x; 1.0712x over previous
import jax
import jax.numpy as jnp
from jax.experimental import pallas as pl
from jax.experimental.pallas import tpu as pltpu


def _moe_block_kernel(x_ref, noise_ref, gate_ref, w_ref, b_ref, out_ref,
                      wb_ref):
    @pl.when(pl.program_id(0) == 0)
    def _cast_weights():
        wb_ref[...] = w_ref[...].astype(jnp.bfloat16)

    xt = x_ref[...]
    logits = jnp.dot(xt, gate_ref[...], preferred_element_type=jnp.float32)
    g = jax.nn.softmax(logits, axis=-1)
    xp = (xt + noise_ref[...]).astype(jnp.bfloat16)
    e_total = b_ref.shape[0]
    d = xp.shape[1]
    acc = jnp.zeros(out_ref.shape, jnp.float32)
    for e in range(e_total):
        ye = jnp.dot(xp, wb_ref[e * d:(e + 1) * d, :],
                     preferred_element_type=jnp.float32)
        acc = acc + g[:, e:e + 1] * (ye + b_ref[e:e + 1, :])
    out_ref[...] = acc


def kernel(x, gate, W, b, noise):
    B, N, D = x.shape
    E = gate.shape[1]
    O = W.shape[2]
    T = B * N
    TT = 1024
    nb = N // TT
    xf = x.reshape(T, D)
    Wf = W.reshape(E * D, O)
    bb = b.astype(jnp.bfloat16)
    out = pl.pallas_call(
        _moe_block_kernel,
        grid=(nb * B,),
        in_specs=[
            pl.BlockSpec((TT, D), lambda i: ((i % B) * nb + i // B, 0)),
            pl.BlockSpec((TT, D), lambda i: (i // B, 0)),
            pl.BlockSpec((D, E), lambda i: (0, 0)),
            pl.BlockSpec((E * D, O), lambda i: (0, 0)),
            pl.BlockSpec((E, O), lambda i: (0, 0)),
        ],
        out_specs=pl.BlockSpec((TT, O), lambda i: ((i % B) * nb + i // B, 0)),
        out_shape=jax.ShapeDtypeStruct((T, O), jnp.float32),
        scratch_shapes=[pltpu.VMEM((E * D, O), jnp.bfloat16)],
    )(xf, noise, gate, Wf, bb)
    return out.reshape(B, N, O)
